# baseline (speedup 1.0000x reference)
import functools

import jax
import jax.numpy as jnp
from jax import lax
from jax.experimental import pallas as pl
from jax.experimental.pallas import tpu as pltpu
from jax.experimental.pallas import tpu_sc as plsc

_B = 4096
_L = 200
_D = 64
_N = _B * _L
_NW = 32
_SUB = 128
_NSUB = 4
_CHUNK = _SUB * _NSUB
_NW_ROWS = _N // _NW
_STEPS = _NW_ROWS // _CHUNK

_mesh = plsc.VectorSubcoreMesh(core_axis_name="c", subcore_axis_name="s")


@functools.partial(
    pl.kernel,
    mesh=_mesh,
    out_type=jax.ShapeDtypeStruct((_N, _D), jnp.float32),
    scratch_types=[
        pltpu.VMEM((_NSUB, _SUB), jnp.int32),
        pltpu.VMEM((_CHUNK, _D), jnp.float32),
        pltpu.SemaphoreType.DMA,
    ],
    compiler_params=pltpu.CompilerParams(use_tc_tiling_on_sc=False),
)
def _embed(xs_hbm, w_hbm, out_hbm, idx_v, rows_v, sem):
    wid = lax.axis_index("s") * 2 + lax.axis_index("c")
    idx_row0 = wid * (_NW_ROWS // _SUB)
    out0 = wid * _NW_ROWS

    def step(i, carry):
        pltpu.sync_copy(xs_hbm.at[pl.ds(idx_row0 + i * _NSUB, _NSUB)], idx_v)
        cps = [
            pltpu.async_copy(
                w_hbm.at[idx_v.at[j]],
                rows_v.at[pl.ds(j * _SUB, _SUB)],
                sem,
            )
            for j in range(_NSUB)
        ]
        for cp in cps:
            cp.wait()
        pltpu.sync_copy(rows_v, out_hbm.at[pl.ds(out0 + i * _CHUNK, _CHUNK)])
        return carry

    lax.fori_loop(0, _STEPS, step, 0)


def kernel(xs, W):
    xs2 = xs.reshape(_N // _SUB, _SUB).astype(jnp.int32)
    out = _embed(xs2, W)
    return out.reshape(_B, _L, _D)


# speedup vs baseline: 1.0472x; 1.0472x over previous
import functools

import jax
import jax.numpy as jnp
from jax import lax
from jax.experimental import pallas as pl
from jax.experimental.pallas import tpu as pltpu
from jax.experimental.pallas import tpu_sc as plsc

_B = 4096
_L = 200
_D = 64
_N = _B * _L
_NW = 32
_SUB = 128
_NSUB = 4
_CHUNK = _SUB * _NSUB
_NW_ROWS = _N // _NW
_IDX_ROWS = _NW_ROWS // _SUB
_STEPS = _NW_ROWS // _CHUNK

_mesh = plsc.VectorSubcoreMesh(core_axis_name="c", subcore_axis_name="s")


@functools.partial(
    pl.kernel,
    mesh=_mesh,
    out_type=jax.ShapeDtypeStruct((_N, _D), jnp.float32),
    scratch_types=[
        pltpu.VMEM((_IDX_ROWS, _SUB), jnp.int32),
        pltpu.VMEM((_CHUNK, _D), jnp.float32),
        pltpu.VMEM((_CHUNK, _D), jnp.float32),
        pltpu.SemaphoreType.DMA,
        pltpu.SemaphoreType.DMA,
        pltpu.SemaphoreType.DMA,
        pltpu.SemaphoreType.DMA,
    ],
    compiler_params=pltpu.CompilerParams(use_tc_tiling_on_sc=False),
)
def _embed(xs_hbm, w_hbm, out_hbm, idx_v, buf0, buf1, sg0, sg1, sw0, sw1):
    wid = lax.axis_index("s") * 2 + lax.axis_index("c")
    out0 = wid * _NW_ROWS

    pltpu.sync_copy(xs_hbm.at[pl.ds(wid * _IDX_ROWS, _IDX_ROWS)], idx_v)

    def gathers(chunk, buf, sem):
        return [
            pltpu.async_copy(
                w_hbm.at[idx_v.at[chunk * _NSUB + j]],
                buf.at[pl.ds(j * _SUB, _SUB)],
                sem,
            )
            for j in range(_NSUB)
        ]

    def wait_gathers(buf, sem):
        for j in range(_NSUB):
            pltpu.make_async_copy(
                w_hbm.at[idx_v.at[j]], buf.at[pl.ds(j * _SUB, _SUB)], sem
            ).wait()

    def write_out(chunk, buf, sem):
        return pltpu.async_copy(
            buf, out_hbm.at[pl.ds(out0 + chunk * _CHUNK, _CHUNK)], sem
        )

    def wait_write(buf, sem):
        pltpu.make_async_copy(
            buf, out_hbm.at[pl.ds(out0, _CHUNK)], sem
        ).wait()

    gathers(0, buf0, sg0)
    gathers(1, buf1, sg1)

    def half(i, buf, sg, sw):
        wait_gathers(buf, sg)
        write_out(i, buf, sw)
        wait_write(buf, sw)
        gathers(i + 2, buf, sg)

    def body(k, carry):
        half(2 * k, buf0, sg0, sw0)
        half(2 * k + 1, buf1, sg1, sw1)
        return carry

    lax.fori_loop(0, _STEPS // 2 - 1, body, 0)

    wait_gathers(buf0, sg0)
    write_out(_STEPS - 2, buf0, sw0)
    wait_gathers(buf1, sg1)
    write_out(_STEPS - 1, buf1, sw1)
    wait_write(buf0, sw0)
    wait_write(buf1, sw1)


def kernel(xs, W):
    xs2 = xs.reshape(_N // _SUB, _SUB).astype(jnp.int32)
    out = _embed(xs2, W)
    return out.reshape(_B, _L, _D)
